# Initial kernel scaffold; baseline (speedup 1.0000x reference)
import jax
import jax.numpy as jnp
from jax.experimental import pallas as pl


def kernel(x, edge_index, W1, b1, W2, b2, W3, b3, g1, be1, g2, be2, Wc1, bc1, Wc2, bc2):
    raise NotImplementedError("write your pallas kernel here")


# speedup vs baseline: 7.5208x; 7.5208x over previous
import functools

import jax
import jax.numpy as jnp
from jax import lax
from jax.experimental import pallas as pl
from jax.experimental.pallas import tpu as pltpu
from jax.experimental.pallas import tpu_sc as plsc

N = 10000
D = 128
EPS = 1e-5

NPAD = 10240
NW = 32
CH = 128
NSTEP = 80
EPW = NSTEP * CH
EPAD = NW * EPW
RPT = NPAD // 16

_mesh = plsc.VectorSubcoreMesh(core_axis_name="c", subcore_axis_name="s")


def _sc_deg_body(dst_hbm, ones_hbm, zeros_hbm, out_hbm, acc_sh, dst_slab, ones_v):
    c = lax.axis_index("c")
    s = lax.axis_index("s")
    pltpu.sync_copy(zeros_hbm.at[pl.ds(s * RPT, RPT)], acc_sh.at[pl.ds(s * RPT, RPT)])
    pltpu.sync_copy(ones_hbm, ones_v)
    w = s * 2 + c
    pltpu.sync_copy(dst_hbm.at[w], dst_slab)
    plsc.subcore_barrier()

    def step(j, carry):
        pltpu.sync_copy(ones_v, acc_sh.at[dst_slab.at[j]], add=True)
        return carry

    lax.fori_loop(0, NSTEP, step, 0)
    plsc.subcore_barrier()
    pltpu.sync_copy(acc_sh.at[pl.ds(s * RPT, RPT)], out_hbm.at[c, pl.ds(s * RPT, RPT)])


def _sc_deg(dst3, ones16, zeros16):
    return pl.kernel(
        _sc_deg_body,
        out_type=jax.ShapeDtypeStruct((2, NPAD, 16), jnp.float32),
        mesh=_mesh,
        scratch_types=[
            pltpu.VMEM_SHARED((NPAD, 16), jnp.float32),
            pltpu.VMEM((NSTEP, CH), jnp.int32),
            pltpu.VMEM((CH, 16), jnp.float32),
        ],
    )(dst3, ones16, zeros16)


def _sc_agg_body(y_hbm, src_hbm, dst_hbm, zeros_hbm, out_hbm,
                 acc_sh, src_slab, dst_slab, rows_v, sem):
    c = lax.axis_index("c")
    s = lax.axis_index("s")
    pltpu.sync_copy(zeros_hbm.at[pl.ds(s * RPT, RPT)], acc_sh.at[pl.ds(s * RPT, RPT)])
    w = s * 2 + c
    pltpu.sync_copy(src_hbm.at[w], src_slab)
    pltpu.sync_copy(dst_hbm.at[w], dst_slab)
    plsc.subcore_barrier()

    def step(j, carry):
        pltpu.async_copy(y_hbm.at[src_slab.at[j]], rows_v, sem).wait()
        pltpu.sync_copy(rows_v, acc_sh.at[dst_slab.at[j]], add=True)
        return carry

    lax.fori_loop(0, NSTEP, step, 0)
    plsc.subcore_barrier()
    pltpu.sync_copy(acc_sh.at[pl.ds(s * RPT, RPT)], out_hbm.at[c, pl.ds(s * RPT, RPT)])


def _sc_agg(y, src3, dst3, zeros):
    return pl.kernel(
        _sc_agg_body,
        out_type=jax.ShapeDtypeStruct((2, NPAD, D), jnp.float32),
        mesh=_mesh,
        scratch_types=[
            pltpu.VMEM_SHARED((NPAD, D), jnp.float32),
            pltpu.VMEM((NSTEP, CH), jnp.int32),
            pltpu.VMEM((NSTEP, CH), jnp.int32),
            pltpu.VMEM((CH, D), jnp.float32),
            pltpu.SemaphoreType.DMA,
        ],
    )(y, src3, dst3, zeros)


R = 2048
_GRID = NPAD // R


def _dot(a, b):
    return lax.dot_general(a, b, (((1,), (0,)), ((), ())),
                           precision=lax.Precision.HIGHEST,
                           preferred_element_type=jnp.float32)


def _k0_body(x_ref, w_ref, p0_ref, p1_ref, y_ref, dv_ref):
    deg = p0_ref[...] + p1_ref[...] + 1.0
    dv = lax.rsqrt(deg)
    dv_ref[...] = dv
    xw = _dot(x_ref[...], w_ref[...])
    y_ref[...] = xw * dv[:, 0:1]


def _k0(xpad, w1, p0, p1):
    return pl.pallas_call(
        _k0_body,
        grid=(_GRID,),
        in_specs=[
            pl.BlockSpec((R, D), lambda j: (j, 0)),
            pl.BlockSpec((D, D), lambda j: (0, 0)),
            pl.BlockSpec((R, 16), lambda j: (j, 0)),
            pl.BlockSpec((R, 16), lambda j: (j, 0)),
        ],
        out_specs=[
            pl.BlockSpec((R, D), lambda j: (j, 0)),
            pl.BlockSpec((R, 16), lambda j: (j, 0)),
        ],
        out_shape=[
            jax.ShapeDtypeStruct((NPAD, D), jnp.float32),
            jax.ShapeDtypeStruct((NPAD, 16), jnp.float32),
        ],
    )(xpad, w1, p0, p1)


def _b1_body(pa_ref, pb_ref, y_ref, dv_ref, b_ref, t_ref, s1_ref, s2_ref):
    j = pl.program_id(0)
    dv = dv_ref[:, 0:1]
    t = dv * (pa_ref[...] + pb_ref[...] + y_ref[...]) + b_ref[...]
    t_ref[...] = t
    rows = lax.broadcasted_iota(jnp.int32, (R, 1), 0) + j * R
    tm = jnp.where(rows < N, t, 0.0)
    s1 = jnp.sum(tm, axis=0, keepdims=True)
    s2 = jnp.sum(tm * tm, axis=0, keepdims=True)

    @pl.when(j == 0)
    def _():
        s1_ref[...] = s1
        s2_ref[...] = s2

    @pl.when(j > 0)
    def _():
        s1_ref[...] += s1
        s2_ref[...] += s2


def _b1(pa, pb, y, dv16, b):
    return pl.pallas_call(
        _b1_body,
        grid=(_GRID,),
        in_specs=[
            pl.BlockSpec((R, D), lambda j: (j, 0)),
            pl.BlockSpec((R, D), lambda j: (j, 0)),
            pl.BlockSpec((R, D), lambda j: (j, 0)),
            pl.BlockSpec((R, 16), lambda j: (j, 0)),
            pl.BlockSpec((1, D), lambda j: (0, 0)),
        ],
        out_specs=[
            pl.BlockSpec((R, D), lambda j: (j, 0)),
            pl.BlockSpec((1, D), lambda j: (0, 0)),
            pl.BlockSpec((1, D), lambda j: (0, 0)),
        ],
        out_shape=[
            jax.ShapeDtypeStruct((NPAD, D), jnp.float32),
            jax.ShapeDtypeStruct((1, D), jnp.float32),
            jax.ShapeDtypeStruct((1, D), jnp.float32),
        ],
    )(pa, pb, y, dv16, b)


def _b2_body(t_ref, s1_ref, s2_ref, g_ref, be_ref, w_ref, dv_ref, y_ref):
    mu = s1_ref[...] * (1.0 / N)
    var = s2_ref[...] * (1.0 / N) - mu * mu
    sc = g_ref[...] * lax.rsqrt(var + EPS)
    h = jnp.maximum((t_ref[...] - mu) * sc + be_ref[...], 0.0)
    xw = _dot(h, w_ref[...])
    y_ref[...] = xw * dv_ref[:, 0:1]


def _b2(t, s1, s2, g, be, wn, dv16):
    return pl.pallas_call(
        _b2_body,
        grid=(_GRID,),
        in_specs=[
            pl.BlockSpec((R, D), lambda j: (j, 0)),
            pl.BlockSpec((1, D), lambda j: (0, 0)),
            pl.BlockSpec((1, D), lambda j: (0, 0)),
            pl.BlockSpec((1, D), lambda j: (0, 0)),
            pl.BlockSpec((1, D), lambda j: (0, 0)),
            pl.BlockSpec((D, D), lambda j: (0, 0)),
            pl.BlockSpec((R, 16), lambda j: (j, 0)),
        ],
        out_specs=[pl.BlockSpec((R, D), lambda j: (j, 0))],
        out_shape=[jax.ShapeDtypeStruct((NPAD, D), jnp.float32)],
    )(t, s1, s2, g, be, wn, dv16)[0]


def _k2_body(pa_ref, pb_ref, y_ref, dv_ref, b_ref, wc1_ref, bc1_ref,
             wc2_ref, bc2_ref, o_ref):
    t = dv_ref[:, 0:1] * (pa_ref[...] + pb_ref[...] + y_ref[...]) + b_ref[...]
    h = jnp.maximum(_dot(t, wc1_ref[...]) + bc1_ref[...], 0.0)
    logits = _dot(h, wc2_ref[...]) + bc2_ref[...]
    mx = jnp.max(logits, axis=1, keepdims=True)
    ex = jnp.exp(logits - mx)
    lse = jnp.log(jnp.sum(ex, axis=1, keepdims=True)) + mx
    o_ref[...] = logits - lse


def _k2(pa, pb, y, dv16, b3, wc1, bc1, wc2, bc2):
    return pl.pallas_call(
        _k2_body,
        grid=(_GRID,),
        in_specs=[
            pl.BlockSpec((R, D), lambda j: (j, 0)),
            pl.BlockSpec((R, D), lambda j: (j, 0)),
            pl.BlockSpec((R, D), lambda j: (j, 0)),
            pl.BlockSpec((R, 16), lambda j: (j, 0)),
            pl.BlockSpec((1, D), lambda j: (0, 0)),
            pl.BlockSpec((D, 64), lambda j: (0, 0)),
            pl.BlockSpec((1, 64), lambda j: (0, 0)),
            pl.BlockSpec((64, 3), lambda j: (0, 0)),
            pl.BlockSpec((1, 3), lambda j: (0, 0)),
        ],
        out_specs=[pl.BlockSpec((R, 3), lambda j: (j, 0))],
        out_shape=[jax.ShapeDtypeStruct((NPAD, 3), jnp.float32)],
    )(pa, pb, y, dv16, b3, wc1, bc1, wc2, bc2)[0]


def kernel(x, edge_index, W1, b1, W2, b2, W3, b3, g1, be1, g2, be2,
           Wc1, bc1, Wc2, bc2):
    e = edge_index.shape[1]
    pad_e = EPAD - e
    src = jnp.concatenate([edge_index[0], jnp.zeros((pad_e,), jnp.int32)])
    dst = jnp.concatenate([edge_index[1], jnp.full((pad_e,), N, jnp.int32)])
    src3 = src.reshape(NW, NSTEP, CH)
    dst3 = dst.reshape(NW, NSTEP, CH)

    xpad = jnp.pad(x, ((0, NPAD - N), (0, 0)))
    zeros = jnp.zeros((NPAD, D), jnp.float32)
    zeros16 = jnp.zeros((NPAD, 16), jnp.float32)
    ones16 = jnp.ones((CH, 16), jnp.float32)

    b1r = b1.reshape(1, D)
    b2r = b2.reshape(1, D)
    b3r = b3.reshape(1, D)
    g1r = g1.reshape(1, D)
    be1r = be1.reshape(1, D)
    g2r = g2.reshape(1, D)
    be2r = be2.reshape(1, D)
    bc1r = bc1.reshape(1, 64)
    bc2r = bc2.reshape(1, 3)

    degp = _sc_deg(dst3, ones16, zeros16)

    y1, dv16 = _k0(xpad, W1, degp[0], degp[1])

    q = _sc_agg(y1, src3, dst3, zeros)
    t1, s1a, s2a = _b1(q[0], q[1], y1, dv16, b1r)
    y2 = _b2(t1, s1a, s2a, g1r, be1r, W2, dv16)

    r = _sc_agg(y2, src3, dst3, zeros)
    t2, s1b, s2b = _b1(r[0], r[1], y2, dv16, b2r)
    y3 = _b2(t2, s1b, s2b, g2r, be2r, W3, dv16)

    u = _sc_agg(y3, src3, dst3, zeros)
    out = _k2(u[0], u[1], y3, dv16, b3r, Wc1, bc1r, Wc2, bc2r)
    return out[:N]


# speedup vs baseline: 7.7230x; 1.0269x over previous
import jax
import jax.numpy as jnp
from jax import lax
from jax.experimental import pallas as pl
from jax.experimental.pallas import tpu as pltpu
from jax.experimental.pallas import tpu_sc as plsc

N = 10000
D = 128
EPS = 1e-5

NACC = 10240
NPAD = 10240
NW = 32
CH = 128
NSTEP = 80
HSTEP = NSTEP // 2
EPW = NSTEP * CH
EPAD = NW * EPW
RPT = NACC // 16

_mesh = plsc.VectorSubcoreMesh(core_axis_name="c", subcore_axis_name="s")


def _sc_deg_body(idx_hbm, ones_hbm, zeros_hbm, out_hbm, acc_sh, slab, ones_v):
    c = lax.axis_index("c")
    s = lax.axis_index("s")
    pltpu.sync_copy(zeros_hbm.at[pl.ds(s * RPT, RPT)], acc_sh.at[pl.ds(s * RPT, RPT)])
    pltpu.sync_copy(ones_hbm, ones_v)
    w = s * 2 + c
    pltpu.sync_copy(idx_hbm.at[w], slab)
    plsc.subcore_barrier()

    def step(j, carry):
        pltpu.sync_copy(ones_v, acc_sh.at[slab.at[2 * j + 1]], add=True)
        return carry

    lax.fori_loop(0, NSTEP, step, 0)
    plsc.subcore_barrier()
    pltpu.sync_copy(acc_sh.at[pl.ds(s * RPT, RPT)], out_hbm.at[c, pl.ds(s * RPT, RPT)])


def _sc_deg(idx3, ones16, zeros16):
    return pl.kernel(
        _sc_deg_body,
        out_type=jax.ShapeDtypeStruct((2, NPAD, 16), jnp.float32),
        mesh=_mesh,
        scratch_types=[
            pltpu.VMEM_SHARED((NACC, 16), jnp.float32),
            pltpu.VMEM((NSTEP * 2, CH), jnp.int32),
            pltpu.VMEM((CH, 16), jnp.float32),
        ],
    )(idx3, ones16, zeros16)


def _sc_agg_body(y_hbm, idx_hbm, zeros_hbm, out_hbm,
                 acc_sh, slab, rows, rsems):
    c = lax.axis_index("c")
    s = lax.axis_index("s")
    w = s * 2 + c

    pltpu.sync_copy(zeros_hbm.at[pl.ds(s * RPT, RPT)], acc_sh.at[pl.ds(s * RPT, RPT)])
    plsc.subcore_barrier()

    for hb in range(2):
        pltpu.sync_copy(idx_hbm.at[w, pl.ds(hb * 2 * HSTEP, 2 * HSTEP)], slab)
        pltpu.async_copy(y_hbm.at[slab.at[0]], rows[0], rsems[0])

        def step(i2, carry):
            for b in range(2):
                l = i2 * 2 + b
                p = b
                pn = 1 - b

                @pl.when(l + 1 < HSTEP)
                def _():
                    pltpu.async_copy(
                        y_hbm.at[slab.at[2 * (l + 1)]], rows[pn], rsems[pn])

                pltpu.make_async_copy(
                    y_hbm.at[slab.at[2 * l]], rows[p], rsems[p]).wait()
                pltpu.sync_copy(rows[p], acc_sh.at[slab.at[2 * l + 1]], add=True)
            return carry

        lax.fori_loop(0, HSTEP // 2, step, 0)

    plsc.subcore_barrier()
    pltpu.sync_copy(acc_sh.at[pl.ds(s * RPT, RPT)], out_hbm.at[c, pl.ds(s * RPT, RPT)])


def _sc_agg(y, idx3, zeros):
    return pl.kernel(
        _sc_agg_body,
        out_type=jax.ShapeDtypeStruct((2, NPAD, D), jnp.float32),
        mesh=_mesh,
        scratch_types=[
            pltpu.VMEM_SHARED((NACC, D), jnp.float32),
            pltpu.VMEM((2 * HSTEP, CH), jnp.int32),
            [pltpu.VMEM((CH, D), jnp.float32) for _ in range(2)],
            [pltpu.SemaphoreType.DMA for _ in range(2)],
        ],
    )(y, idx3, zeros)


R = 2048
_GRID = NPAD // R


def _dot(a, b):
    return lax.dot_general(a, b, (((1,), (0,)), ((), ())),
                           precision=lax.Precision.HIGHEST,
                           preferred_element_type=jnp.float32)


def _rowmask(j):
    rows = lax.broadcasted_iota(jnp.int32, (R, 1), 0) + j * R
    return rows < N


def _k0_body(x_ref, w_ref, p0_ref, p1_ref, pc_ref, y_ref, dv_ref):
    deg = p0_ref[...] + p1_ref[...] + 1.0 - pc_ref[...]
    dv = lax.rsqrt(deg)
    dv_ref[...] = dv
    xw = _dot(x_ref[...], w_ref[...])
    y_ref[...] = jnp.where(_rowmask(pl.program_id(0)), xw * dv[:, 0:1], 0.0)


def _k0(xpad, w1, p0, p1, padc):
    return pl.pallas_call(
        _k0_body,
        grid=(_GRID,),
        in_specs=[
            pl.BlockSpec((R, D), lambda j: (j, 0)),
            pl.BlockSpec((D, D), lambda j: (0, 0)),
            pl.BlockSpec((R, 16), lambda j: (j, 0)),
            pl.BlockSpec((R, 16), lambda j: (j, 0)),
            pl.BlockSpec((R, 16), lambda j: (j, 0)),
        ],
        out_specs=[
            pl.BlockSpec((R, D), lambda j: (j, 0)),
            pl.BlockSpec((R, 16), lambda j: (j, 0)),
        ],
        out_shape=[
            jax.ShapeDtypeStruct((NPAD, D), jnp.float32),
            jax.ShapeDtypeStruct((NPAD, 16), jnp.float32),
        ],
    )(xpad, w1, p0, p1, padc)


def _b1_body(pa_ref, pb_ref, y_ref, dv_ref, b_ref, t_ref, s1_ref, s2_ref):
    j = pl.program_id(0)
    dv = dv_ref[:, 0:1]
    t = dv * (pa_ref[...] + pb_ref[...] + y_ref[...]) + b_ref[...]
    t_ref[...] = t
    tm = jnp.where(_rowmask(j), t, 0.0)
    s1 = jnp.sum(tm, axis=0, keepdims=True)
    s2 = jnp.sum(tm * tm, axis=0, keepdims=True)

    @pl.when(j == 0)
    def _():
        s1_ref[...] = s1
        s2_ref[...] = s2

    @pl.when(j > 0)
    def _():
        s1_ref[...] += s1
        s2_ref[...] += s2


def _b1(pa, pb, y, dv16, b):
    return pl.pallas_call(
        _b1_body,
        grid=(_GRID,),
        in_specs=[
            pl.BlockSpec((R, D), lambda j: (j, 0)),
            pl.BlockSpec((R, D), lambda j: (j, 0)),
            pl.BlockSpec((R, D), lambda j: (j, 0)),
            pl.BlockSpec((R, 16), lambda j: (j, 0)),
            pl.BlockSpec((1, D), lambda j: (0, 0)),
        ],
        out_specs=[
            pl.BlockSpec((R, D), lambda j: (j, 0)),
            pl.BlockSpec((1, D), lambda j: (0, 0)),
            pl.BlockSpec((1, D), lambda j: (0, 0)),
        ],
        out_shape=[
            jax.ShapeDtypeStruct((NPAD, D), jnp.float32),
            jax.ShapeDtypeStruct((1, D), jnp.float32),
            jax.ShapeDtypeStruct((1, D), jnp.float32),
        ],
    )(pa, pb, y, dv16, b)


def _b2_body(t_ref, s1_ref, s2_ref, g_ref, be_ref, w_ref, dv_ref, y_ref):
    mu = s1_ref[...] * (1.0 / N)
    var = s2_ref[...] * (1.0 / N) - mu * mu
    sc = g_ref[...] * lax.rsqrt(var + EPS)
    h = jnp.maximum((t_ref[...] - mu) * sc + be_ref[...], 0.0)
    xw = _dot(h, w_ref[...])
    y_ref[...] = jnp.where(_rowmask(pl.program_id(0)), xw * dv_ref[:, 0:1], 0.0)


def _b2(t, s1, s2, g, be, wn, dv16):
    return pl.pallas_call(
        _b2_body,
        grid=(_GRID,),
        in_specs=[
            pl.BlockSpec((R, D), lambda j: (j, 0)),
            pl.BlockSpec((1, D), lambda j: (0, 0)),
            pl.BlockSpec((1, D), lambda j: (0, 0)),
            pl.BlockSpec((1, D), lambda j: (0, 0)),
            pl.BlockSpec((1, D), lambda j: (0, 0)),
            pl.BlockSpec((D, D), lambda j: (0, 0)),
            pl.BlockSpec((R, 16), lambda j: (j, 0)),
        ],
        out_specs=[pl.BlockSpec((R, D), lambda j: (j, 0))],
        out_shape=[jax.ShapeDtypeStruct((NPAD, D), jnp.float32)],
    )(t, s1, s2, g, be, wn, dv16)[0]


def _k2_body(pa_ref, pb_ref, y_ref, dv_ref, b_ref, wc1_ref, bc1_ref,
             wc2_ref, bc2_ref, o_ref):
    t = dv_ref[:, 0:1] * (pa_ref[...] + pb_ref[...] + y_ref[...]) + b_ref[...]
    h = jnp.maximum(_dot(t, wc1_ref[...]) + bc1_ref[...], 0.0)
    logits = _dot(h, wc2_ref[...]) + bc2_ref[...]
    mx = jnp.max(logits, axis=1, keepdims=True)
    ex = jnp.exp(logits - mx)
    lse = jnp.log(jnp.sum(ex, axis=1, keepdims=True)) + mx
    o_ref[...] = logits - lse


def _k2(pa, pb, y, dv16, b3, wc1, bc1, wc2, bc2):
    return pl.pallas_call(
        _k2_body,
        grid=(_GRID,),
        in_specs=[
            pl.BlockSpec((R, D), lambda j: (j, 0)),
            pl.BlockSpec((R, D), lambda j: (j, 0)),
            pl.BlockSpec((R, D), lambda j: (j, 0)),
            pl.BlockSpec((R, 16), lambda j: (j, 0)),
            pl.BlockSpec((1, D), lambda j: (0, 0)),
            pl.BlockSpec((D, 64), lambda j: (0, 0)),
            pl.BlockSpec((1, 64), lambda j: (0, 0)),
            pl.BlockSpec((64, 3), lambda j: (0, 0)),
            pl.BlockSpec((1, 3), lambda j: (0, 0)),
        ],
        out_specs=[pl.BlockSpec((R, 3), lambda j: (j, 0))],
        out_shape=[jax.ShapeDtypeStruct((NPAD, 3), jnp.float32)],
    )(pa, pb, y, dv16, b3, wc1, bc1, wc2, bc2)[0]


def kernel(x, edge_index, W1, b1, W2, b2, W3, b3, g1, be1, g2, be2,
           Wc1, bc1, Wc2, bc2):
    e = edge_index.shape[1]
    pad_e = EPAD - e
    src = jnp.concatenate([edge_index[0], jnp.full((pad_e,), N, jnp.int32)])
    dst = jnp.concatenate(
        [edge_index[1], jnp.arange(pad_e, dtype=jnp.int32) % NACC])
    src3 = src.reshape(NW, NSTEP, 1, CH)
    dst3 = dst.reshape(NW, NSTEP, 1, CH)
    idx3 = jnp.concatenate([src3, dst3], axis=2).reshape(NW, NSTEP * 2, CH)

    full, rem = pad_e // NACC, pad_e % NACC
    r = jnp.arange(NPAD)
    padcol = jnp.where(r < NACC, full + (r < rem), 0).astype(jnp.float32)
    padc = jnp.broadcast_to(padcol[:, None], (NPAD, 16))

    xpad = jnp.pad(x, ((0, NPAD - N), (0, 0)))
    zeros = jnp.zeros((NACC, D), jnp.float32)
    zeros16 = jnp.zeros((NACC, 16), jnp.float32)
    ones16 = jnp.ones((CH, 16), jnp.float32)

    b1r = b1.reshape(1, D)
    b2r = b2.reshape(1, D)
    b3r = b3.reshape(1, D)
    g1r = g1.reshape(1, D)
    be1r = be1.reshape(1, D)
    g2r = g2.reshape(1, D)
    be2r = be2.reshape(1, D)
    bc1r = bc1.reshape(1, 64)
    bc2r = bc2.reshape(1, 3)

    degp = _sc_deg(idx3, ones16, zeros16)

    y1, dv16 = _k0(xpad, W1, degp[0], degp[1], padc)

    q = _sc_agg(y1, idx3, zeros)
    t1, s1a, s2a = _b1(q[0], q[1], y1, dv16, b1r)
    y2 = _b2(t1, s1a, s2a, g1r, be1r, W2, dv16)

    u = _sc_agg(y2, idx3, zeros)
    t2, s1b, s2b = _b1(u[0], u[1], y2, dv16, b2r)
    y3 = _b2(t2, s1b, s2b, g2r, be2r, W3, dv16)

    v = _sc_agg(y3, idx3, zeros)
    out = _k2(v[0], v[1], y3, dv16, b3r, Wc1, bc1r, Wc2, bc2r)
    return out[:N]
